# Initial kernel scaffold; baseline (speedup 1.0000x reference)
import jax
import jax.numpy as jnp
from jax.experimental import pallas as pl


def kernel(embeddings, head_idx, head_e, tail_idx, tail_e, adj_src, adj_dst, relation_kernel, self_kernel):
    raise NotImplementedError("write your pallas kernel here")


# speedup vs baseline: 5.5364x; 5.5364x over previous
import functools

import jax
import jax.numpy as jnp
from jax import lax
from jax.experimental import pallas as pl
from jax.experimental.pallas import tpu as pltpu
from jax.experimental.pallas import tpu_sc as plsc

NC = 2
NS = 16
NW = NC * NS

CHUNK = 128


def _seg_sum_sc(embeddings, adj_src, adj_dst, zeros):
    N, D = embeddings.shape
    R, E = adj_src.shape
    rel_per_core = R // NC
    npt = (N // NS) // 8 * 8
    npt_last = N - (NS - 1) * npt
    nfull = E // CHUNK // NS
    ntail = E // CHUNK - nfull * NS

    mesh = plsc.VectorSubcoreMesh(core_axis_name="c", subcore_axis_name="s")

    @functools.partial(
        pl.kernel,
        out_type=jax.ShapeDtypeStruct((R, N, D), jnp.float32),
        mesh=mesh,
        scratch_types=[
            pltpu.VMEM((CHUNK,), jnp.int32),
            pltpu.VMEM((CHUNK,), jnp.int32),
            pltpu.VMEM((CHUNK, D), jnp.float32),
            pltpu.VMEM_SHARED((N, D), jnp.float32),
            pltpu.SemaphoreType.DMA,
        ],
    )
    def seg_sum(emb, asrc, adst, zer, out, src_i, dst_i, rows, acc, sem):
        c = lax.axis_index("c")
        s = lax.axis_index("s")

        def do_chunk(r, base):
            pltpu.sync_copy(asrc.at[r, pl.ds(base, CHUNK)], src_i)
            pltpu.sync_copy(adst.at[r, pl.ds(base, CHUNK)], dst_i)
            pltpu.async_copy(emb.at[src_i], rows, sem).wait()
            pltpu.sync_copy(rows, acc.at[dst_i], add=True)

        for rloc in range(rel_per_core):
            r = c * rel_per_core + rloc
            @pl.when(s < NS - 1)
            def _():
                pltpu.sync_copy(zer.at[pl.ds(0, npt)],
                                acc.at[pl.ds(s * npt, npt)])

            @pl.when(s == NS - 1)
            def _():
                pltpu.sync_copy(zer.at[pl.ds(0, npt_last)],
                                acc.at[pl.ds((NS - 1) * npt, npt_last)])

            plsc.subcore_barrier()

            @pl.loop(0, nfull)
            def _(j):
                do_chunk(r, (j * NS + s) * CHUNK)

            if ntail:
                @pl.when(s < ntail)
                def _():
                    do_chunk(r, (nfull * NS + s) * CHUNK)

            plsc.subcore_barrier()

            @pl.when(s < NS - 1)
            def _():
                pltpu.sync_copy(acc.at[pl.ds(s * npt, npt)],
                                out.at[r, pl.ds(s * npt, npt)])

            @pl.when(s == NS - 1)
            def _():
                pltpu.sync_copy(acc.at[pl.ds((NS - 1) * npt, npt_last)],
                                out.at[r, pl.ds((NS - 1) * npt, npt_last)])

            plsc.subcore_barrier()

    return seg_sum(embeddings, adj_src, adj_dst, zeros)


def _gather_sc(node_out, head_idx, tail_idx):
    N, D = node_out.shape
    B = head_idx.shape[0]
    per_w = B // NW
    nch = per_w // CHUNK

    mesh = plsc.VectorSubcoreMesh(core_axis_name="c", subcore_axis_name="s")

    @functools.partial(
        pl.kernel,
        out_type=(jax.ShapeDtypeStruct((B, D), jnp.float32),
                  jax.ShapeDtypeStruct((B, D), jnp.float32)),
        mesh=mesh,
        scratch_types=[
            pltpu.VMEM((CHUNK,), jnp.int32),
            pltpu.VMEM((CHUNK, D), jnp.float32),
            pltpu.SemaphoreType.DMA,
        ],
    )
    def gat(node, hidx, tidx, oh, ot, idx_v, rows, sem):
        c = lax.axis_index("c")
        s = lax.axis_index("s")
        w = s * NC + c

        @pl.loop(0, nch)
        def _(j):
            base = w * per_w + j * CHUNK
            pltpu.sync_copy(hidx.at[pl.ds(base, CHUNK)], idx_v)
            pltpu.async_copy(node.at[idx_v], rows, sem).wait()
            pltpu.sync_copy(rows, oh.at[pl.ds(base, CHUNK)])
            pltpu.sync_copy(tidx.at[pl.ds(base, CHUNK)], idx_v)
            pltpu.async_copy(node.at[idx_v], rows, sem).wait()
            pltpu.sync_copy(rows, ot.at[pl.ds(base, CHUNK)])

    return gat(node_out, head_idx, tail_idx)


def _relation_matmul_tc(S, relation_kernel):
    R, N, D = S.shape
    OUT = relation_kernel.shape[-1]
    BN = 2000

    def mm(s_ref, w_ref, o_ref):
        acc = jnp.dot(s_ref[0], w_ref[0], preferred_element_type=jnp.float32)
        for r in range(1, R):
            acc += jnp.dot(s_ref[r], w_ref[r],
                           preferred_element_type=jnp.float32)
        o_ref[...] = acc

    return pl.pallas_call(
        mm,
        grid=(N // BN,),
        in_specs=[
            pl.BlockSpec((R, BN, D), lambda i: (0, i, 0)),
            pl.BlockSpec((R, D, OUT), lambda i: (0, 0, 0)),
        ],
        out_specs=pl.BlockSpec((BN, OUT), lambda i: (i, 0)),
        out_shape=jax.ShapeDtypeStruct((N, OUT), jnp.float32),
    )(S, relation_kernel)


def _final_tc(head_e, tail_e, gath_h, gath_t, self_kernel):
    B, D = head_e.shape
    OUT = self_kernel.shape[-1]
    BB = 2048

    def fin(he, te, gh, gt, sk, oh, ot):
        oh[...] = jax.nn.sigmoid(
            jnp.dot(he[...], sk[...], preferred_element_type=jnp.float32)
            + gh[...])
        ot[...] = jax.nn.sigmoid(
            jnp.dot(te[...], sk[...], preferred_element_type=jnp.float32)
            + gt[...])

    return pl.pallas_call(
        fin,
        grid=(B // BB,),
        in_specs=[
            pl.BlockSpec((BB, D), lambda i: (i, 0)),
            pl.BlockSpec((BB, D), lambda i: (i, 0)),
            pl.BlockSpec((BB, OUT), lambda i: (i, 0)),
            pl.BlockSpec((BB, OUT), lambda i: (i, 0)),
            pl.BlockSpec((D, OUT), lambda i: (0, 0)),
        ],
        out_specs=(pl.BlockSpec((BB, OUT), lambda i: (i, 0)),
                   pl.BlockSpec((BB, OUT), lambda i: (i, 0))),
        out_shape=(jax.ShapeDtypeStruct((B, OUT), jnp.float32),
                   jax.ShapeDtypeStruct((B, OUT), jnp.float32)),
    )(head_e, tail_e, gath_h, gath_t, self_kernel)


def kernel(embeddings, head_idx, head_e, tail_idx, tail_e, adj_src, adj_dst,
           relation_kernel, self_kernel):
    N, D = embeddings.shape
    zeros = jnp.zeros((N - (NS - 1) * ((N // NS) // 8 * 8), D), jnp.float32)
    S = _seg_sum_sc(embeddings, adj_src, adj_dst, zeros)
    node_out = _relation_matmul_tc(S, relation_kernel)
    gath_h, gath_t = _gather_sc(node_out, head_idx, tail_idx)
    return _final_tc(head_e, tail_e, gath_h, gath_t, self_kernel)


# speedup vs baseline: 8.6606x; 1.5643x over previous
import functools

import jax
import jax.numpy as jnp
from jax import lax
from jax.experimental import pallas as pl
from jax.experimental.pallas import tpu as pltpu
from jax.experimental.pallas import tpu_sc as plsc

NC = 2
NS = 16
NW = NC * NS

CHUNK = 128


def _seg_sum_sc(embeddings, adj_src, adj_dst, zeros):
    N, D = embeddings.shape
    R, NCHT, _ = adj_src.shape
    rel_per_core = R // NC
    npt = (N // NS) // 8 * 8
    npt_last = N - (NS - 1) * npt
    NCH = (-(-NCHT // NS) + 7) // 8 * 8
    NCH_LAST = NCHT - (NS - 1) * NCH
    assert 0 < NCH_LAST <= NCH
    NBUF = 2

    mesh = plsc.VectorSubcoreMesh(core_axis_name="c", subcore_axis_name="s")

    @functools.partial(
        pl.kernel,
        out_type=jax.ShapeDtypeStruct((R, N, D), jnp.float32),
        mesh=mesh,
        scratch_types=[
            pltpu.VMEM((NCH, CHUNK), jnp.int32),
            pltpu.VMEM((NCH, CHUNK), jnp.int32),
            pltpu.VMEM((NBUF, CHUNK, D), jnp.float32),
            pltpu.VMEM_SHARED((N, D), jnp.float32),
            pltpu.SemaphoreType.DMA,
            pltpu.SemaphoreType.DMA,
        ],
    )
    def seg_sum(emb, asrc, adst, zer, out, src_i, dst_i, rows, acc,
                sem_g, sem_s):
        c = lax.axis_index("c")
        s = lax.axis_index("s")

        def fire_gather(j):
            pltpu.async_copy(emb.at[src_i.at[j]],
                             rows.at[lax.rem(j, NBUF)], sem_g)

        def wait_gather():
            pltpu.make_async_copy(emb.at[src_i.at[0]],
                                  rows.at[0], sem_g).wait()

        def wait_scatter():
            pltpu.make_async_copy(rows.at[0],
                                  acc.at[pl.ds(0, CHUNK)], sem_s).wait()

        for rloc in range(rel_per_core):
            r = c * rel_per_core + rloc
            @pl.when(s < NS - 1)
            def _():
                pltpu.sync_copy(zer.at[pl.ds(0, npt)],
                                acc.at[pl.ds(s * npt, npt)])

            @pl.when(s == NS - 1)
            def _():
                pltpu.sync_copy(zer.at[pl.ds(0, npt_last)],
                                acc.at[pl.ds((NS - 1) * npt, npt_last)])

            @pl.when(s < NS - 1)
            def _():
                pltpu.sync_copy(asrc.at[r, pl.ds(s * NCH, NCH)],
                                src_i.at[pl.ds(0, NCH)])
                pltpu.sync_copy(adst.at[r, pl.ds(s * NCH, NCH)],
                                dst_i.at[pl.ds(0, NCH)])

            @pl.when(s == NS - 1)
            def _():
                pltpu.sync_copy(
                    asrc.at[r, pl.ds((NS - 1) * NCH, NCH_LAST)],
                    src_i.at[pl.ds(0, NCH_LAST)])
                pltpu.sync_copy(
                    adst.at[r, pl.ds((NS - 1) * NCH, NCH_LAST)],
                    dst_i.at[pl.ds(0, NCH_LAST)])

            nch_t = jnp.where(s < NS - 1, NCH, NCH_LAST)

            for k in range(NBUF - 1):
                fire_gather(k)

            plsc.subcore_barrier()

            @pl.loop(0, nch_t)
            def _(j):
                wait_gather()

                @pl.when(j >= 1)
                def _():
                    wait_scatter()

                @pl.when(j + NBUF - 1 < nch_t)
                def _():
                    fire_gather(j + NBUF - 1)

                pltpu.async_copy(rows.at[lax.rem(j, NBUF)],
                                 acc.at[dst_i.at[j]], sem_s, add=True)

            wait_scatter()
            plsc.subcore_barrier()

            @pl.when(s < NS - 1)
            def _():
                pltpu.sync_copy(acc.at[pl.ds(s * npt, npt)],
                                out.at[r, pl.ds(s * npt, npt)])

            @pl.when(s == NS - 1)
            def _():
                pltpu.sync_copy(acc.at[pl.ds((NS - 1) * npt, npt_last)],
                                out.at[r, pl.ds((NS - 1) * npt, npt_last)])

            plsc.subcore_barrier()

    return seg_sum(embeddings, adj_src, adj_dst, zeros)


def _gather_sc(node_out, ht_idx):
    N, D = node_out.shape
    ncht = ht_idx.shape[0]
    B = ncht * CHUNK // 2
    nch = ncht // NW
    assert nch * NW == ncht and nch % 8 == 0

    mesh = plsc.VectorSubcoreMesh(core_axis_name="c", subcore_axis_name="s")

    @functools.partial(
        pl.kernel,
        out_type=(jax.ShapeDtypeStruct((B, D), jnp.float32),
                  jax.ShapeDtypeStruct((B, D), jnp.float32)),
        mesh=mesh,
        scratch_types=[
            pltpu.VMEM((nch, CHUNK), jnp.int32),
            pltpu.VMEM((2, CHUNK, D), jnp.float32),
            pltpu.SemaphoreType.DMA,
        ],
    )
    def gat(node, htidx, oh, ot, idx_v, rows, sem):
        c = lax.axis_index("c")
        s = lax.axis_index("s")
        w = s * NC + c

        pltpu.sync_copy(htidx.at[pl.ds(w * nch, nch)], idx_v)
        descs = [None] * nch
        descs[0] = pltpu.async_copy(node.at[idx_v.at[0]], rows.at[0], sem)
        for j in range(nch):
            if j + 1 < nch:
                descs[j + 1] = pltpu.async_copy(
                    node.at[idx_v.at[j + 1]], rows.at[(j + 1) % 2], sem)
            descs[j].wait()

            @pl.when(w < NW // 2)
            def _():
                pltpu.sync_copy(
                    rows.at[j % 2],
                    oh.at[pl.ds((w * nch + j) * CHUNK, CHUNK)])

            @pl.when(w >= NW // 2)
            def _():
                pltpu.sync_copy(
                    rows.at[j % 2],
                    ot.at[pl.ds(((w - NW // 2) * nch + j) * CHUNK, CHUNK)])

    return gat(node_out, ht_idx)


def _relation_matmul_tc(S, relation_kernel):
    R, N, D = S.shape
    OUT = relation_kernel.shape[-1]
    BN = 2000

    def mm(s_ref, w_ref, o_ref):
        acc = jnp.dot(s_ref[0], w_ref[0], preferred_element_type=jnp.float32)
        for r in range(1, R):
            acc += jnp.dot(s_ref[r], w_ref[r],
                           preferred_element_type=jnp.float32)
        o_ref[...] = acc

    return pl.pallas_call(
        mm,
        grid=(N // BN,),
        in_specs=[
            pl.BlockSpec((R, BN, D), lambda i: (0, i, 0)),
            pl.BlockSpec((R, D, OUT), lambda i: (0, 0, 0)),
        ],
        out_specs=pl.BlockSpec((BN, OUT), lambda i: (i, 0)),
        out_shape=jax.ShapeDtypeStruct((N, OUT), jnp.float32),
    )(S, relation_kernel)


def _final_tc(head_e, tail_e, gath_h, gath_t, self_kernel):
    B, D = head_e.shape
    OUT = self_kernel.shape[-1]
    BB = 2048

    def fin(he, te, gh, gt, sk, oh, ot):
        oh[...] = jax.nn.sigmoid(
            jnp.dot(he[...], sk[...], preferred_element_type=jnp.float32)
            + gh[...])
        ot[...] = jax.nn.sigmoid(
            jnp.dot(te[...], sk[...], preferred_element_type=jnp.float32)
            + gt[...])

    return pl.pallas_call(
        fin,
        grid=(B // BB,),
        in_specs=[
            pl.BlockSpec((BB, D), lambda i: (i, 0)),
            pl.BlockSpec((BB, D), lambda i: (i, 0)),
            pl.BlockSpec((BB, OUT), lambda i: (i, 0)),
            pl.BlockSpec((BB, OUT), lambda i: (i, 0)),
            pl.BlockSpec((D, OUT), lambda i: (0, 0)),
        ],
        out_specs=(pl.BlockSpec((BB, OUT), lambda i: (i, 0)),
                   pl.BlockSpec((BB, OUT), lambda i: (i, 0))),
        out_shape=(jax.ShapeDtypeStruct((B, OUT), jnp.float32),
                   jax.ShapeDtypeStruct((B, OUT), jnp.float32)),
    )(head_e, tail_e, gath_h, gath_t, self_kernel)


def kernel(embeddings, head_idx, head_e, tail_idx, tail_e, adj_src, adj_dst,
           relation_kernel, self_kernel):
    N, D = embeddings.shape
    R, E = adj_src.shape
    zeros = jnp.zeros((N - (NS - 1) * ((N // NS) // 8 * 8), D), jnp.float32)
    S = _seg_sum_sc(embeddings,
                    adj_src.reshape(R, E // CHUNK, CHUNK),
                    adj_dst.reshape(R, E // CHUNK, CHUNK), zeros)
    node_out = _relation_matmul_tc(S, relation_kernel)
    ht_idx = jnp.concatenate([head_idx, tail_idx]).reshape(-1, CHUNK)
    gath_h, gath_t = _gather_sc(node_out, ht_idx)
    return _final_tc(head_e, tail_e, gath_h, gath_t, self_kernel)
